# Initial kernel scaffold; baseline (speedup 1.0000x reference)
import jax
import jax.numpy as jnp
from jax.experimental import pallas as pl


def kernel(s_j, v_j, r_ij, nbrs, W_phi, b_phi, W_rbf):
    raise NotImplementedError("write your pallas kernel here")


# speedup vs baseline: 4.9423x; 4.9423x over previous
import functools

import jax
import jax.numpy as jnp
from jax import lax
from jax.experimental import pallas as pl
from jax.experimental.pallas import tpu as pltpu

EPS = 1e-15
N_NODES = 10000
N_EDGES = 320000
FEAT = 128
N_RBF = 20
CUTOFF = 5.0


_BN = 1000


def _phi_body(s_ref, w_ref, b_ref, o_ref):
    o_ref[...] = (
        jnp.dot(s_ref[...], w_ref[...], preferred_element_type=jnp.float32)
        + b_ref[...]
    )


def _compute_phi(s_j, W_phi, b_phi):
    n = s_j.shape[0]
    grid = n // _BN
    return pl.pallas_call(
        _phi_body,
        grid=(grid,),
        in_specs=[
            pl.BlockSpec((_BN, FEAT), lambda i: (i, 0)),
            pl.BlockSpec((FEAT, 3 * FEAT), lambda i: (0, 0)),
            pl.BlockSpec((1, 3 * FEAT), lambda i: (0, 0)),
        ],
        out_specs=pl.BlockSpec((_BN, 3 * FEAT), lambda i: (i, 0)),
        out_shape=jax.ShapeDtypeStruct((n, 3 * FEAT), jnp.float32),
    )(s_j, W_phi, b_phi.reshape(1, -1))


_BE = 1000


def _edge_body(r_ref, phig_ref, vx_ref, vy_ref, vz_ref, freq_ref, wrbf_ref,
               ds_ref, dvx_ref, dvy_ref, dvz_ref):
    r = r_ref[...]
    d2 = (r * r).sum(axis=1, keepdims=True) + 3.0 * EPS
    dist = jnp.sqrt(d2)
    inv = 1.0 / dist
    rbf = jnp.sin(dist * freq_ref[...]) * inv
    w_s = jnp.dot(rbf, wrbf_ref[...], preferred_element_type=jnp.float32,
                  precision=jax.lax.Precision.HIGHEST)
    phig = phig_ref[...]
    sp0 = phig[:, :FEAT] * w_s[:, :FEAT]
    sp1 = phig[:, FEAT:2 * FEAT] * w_s[:, FEAT:2 * FEAT]
    sp2 = phig[:, 2 * FEAT:] * w_s[:, 2 * FEAT:]
    ds_ref[...] = sp1
    ux = r[:, 0:1] * inv
    uy = r[:, 1:2] * inv
    uz = r[:, 2:3] * inv
    dvx_ref[...] = sp2 * ux + sp0 * vx_ref[...]
    dvy_ref[...] = sp2 * uy + sp0 * vy_ref[...]
    dvz_ref[...] = sp2 * uz + sp0 * vz_ref[...]


def _edge_math(r_ij, phi_g, vxg, vyg, vzg, W_rbf):
    e = r_ij.shape[0]
    grid = e // _BE
    freq = jnp.pad(
        jnp.arange(1, N_RBF + 1, dtype=jnp.float32) * (jnp.pi / CUTOFF),
        (0, FEAT - N_RBF),
    ).reshape(1, FEAT)
    wrbf_pad = jnp.pad(W_rbf, ((0, FEAT - N_RBF), (0, 0)))
    fspec = pl.BlockSpec((_BE, FEAT), lambda i: (i, 0))
    out4 = [jax.ShapeDtypeStruct((e, FEAT), jnp.float32)] * 4
    return pl.pallas_call(
        _edge_body,
        grid=(grid,),
        in_specs=[
            pl.BlockSpec((_BE, 3), lambda i: (i, 0)),
            pl.BlockSpec((_BE, 3 * FEAT), lambda i: (i, 0)),
            fspec, fspec, fspec,
            pl.BlockSpec((1, FEAT), lambda i: (0, 0)),
            pl.BlockSpec((FEAT, 3 * FEAT), lambda i: (0, 0)),
        ],
        out_specs=[fspec, fspec, fspec, fspec],
        out_shape=out4,
    )(r_ij, phi_g, vxg, vyg, vzg, freq, wrbf_pad)


def kernel(s_j, v_j, r_ij, nbrs, W_phi, b_phi, W_rbf):
    nbrs = nbrs.astype(jnp.int32)
    src = nbrs[:, 0]
    dst = nbrs[:, 1]
    phi = _compute_phi(s_j, W_phi, b_phi)
    vt = jnp.transpose(v_j, (2, 0, 1))
    phi_g = jnp.take(phi, dst, axis=0)
    vxg = jnp.take(vt[0], dst, axis=0)
    vyg = jnp.take(vt[1], dst, axis=0)
    vzg = jnp.take(vt[2], dst, axis=0)
    ds, dvx, dvy, dvz = _edge_math(r_ij, phi_g, vxg, vyg, vzg, W_rbf)
    n = s_j.shape[0]
    delta_s = jax.ops.segment_sum(ds, src, num_segments=n)
    ovx = jax.ops.segment_sum(dvx, src, num_segments=n)
    ovy = jax.ops.segment_sum(dvy, src, num_segments=n)
    ovz = jax.ops.segment_sum(dvz, src, num_segments=n)
    delta_v = jnp.stack([ovx, ovy, ovz], axis=-1)
    return (delta_s, delta_v)


# speedup vs baseline: 12.5031x; 2.5298x over previous
import functools

import jax
import jax.numpy as jnp
from jax import lax
from jax.experimental import pallas as pl
from jax.experimental.pallas import tpu as pltpu
from jax.experimental.pallas import tpu_sc as plsc

EPS = 1e-15
N_NODES = 10000
N_EDGES = 320000
FEAT = 128
N_RBF = 20
CUTOFF = 5.0


_BN = 1000


def _phi_body(s_ref, w_ref, b_ref, o_ref):
    o_ref[...] = (
        jnp.dot(s_ref[...], w_ref[...], preferred_element_type=jnp.float32)
        + b_ref[...]
    )


def _compute_phi(s_j, W_phi, b_phi):
    n = s_j.shape[0]
    grid = n // _BN
    return pl.pallas_call(
        _phi_body,
        grid=(grid,),
        in_specs=[
            pl.BlockSpec((_BN, FEAT), lambda i: (i, 0)),
            pl.BlockSpec((FEAT, 3 * FEAT), lambda i: (0, 0)),
            pl.BlockSpec((1, 3 * FEAT), lambda i: (0, 0)),
        ],
        out_specs=pl.BlockSpec((_BN, 3 * FEAT), lambda i: (i, 0)),
        out_shape=jax.ShapeDtypeStruct((n, 3 * FEAT), jnp.float32),
    )(s_j, W_phi, b_phi.reshape(1, -1))


_BE = 1000


def _edge_body(r_ref, phig_ref, vx_ref, vy_ref, vz_ref, freq_ref, wrbf_ref,
               ds_ref, dvx_ref, dvy_ref, dvz_ref):
    r = r_ref[...]
    d2 = (r * r).sum(axis=1, keepdims=True) + 3.0 * EPS
    dist = jnp.sqrt(d2)
    inv = 1.0 / dist
    rbf = jnp.sin(dist * freq_ref[...]) * inv
    w_s = jnp.dot(rbf, wrbf_ref[...], preferred_element_type=jnp.float32,
                  precision=jax.lax.Precision.HIGHEST)
    phig = phig_ref[...]
    sp0 = phig[:, :FEAT] * w_s[:, :FEAT]
    sp1 = phig[:, FEAT:2 * FEAT] * w_s[:, FEAT:2 * FEAT]
    sp2 = phig[:, 2 * FEAT:] * w_s[:, 2 * FEAT:]
    ds_ref[...] = sp1
    ux = r[:, 0:1] * inv
    uy = r[:, 1:2] * inv
    uz = r[:, 2:3] * inv
    dvx_ref[...] = sp2 * ux + sp0 * vx_ref[...]
    dvy_ref[...] = sp2 * uy + sp0 * vy_ref[...]
    dvz_ref[...] = sp2 * uz + sp0 * vz_ref[...]


def _edge_math(r_ij, phi_g, vxg, vyg, vzg, W_rbf):
    e = r_ij.shape[0]
    grid = e // _BE
    freq = jnp.pad(
        jnp.arange(1, N_RBF + 1, dtype=jnp.float32) * (jnp.pi / CUTOFF),
        (0, FEAT - N_RBF),
    ).reshape(1, FEAT)
    wrbf_pad = jnp.pad(W_rbf, ((0, FEAT - N_RBF), (0, 0)))
    fspec = pl.BlockSpec((_BE, FEAT), lambda i: (i, 0))
    out4 = [jax.ShapeDtypeStruct((e, FEAT), jnp.float32)] * 4
    return pl.pallas_call(
        _edge_body,
        grid=(grid,),
        in_specs=[
            pl.BlockSpec((_BE, 3), lambda i: (i, 0)),
            pl.BlockSpec((_BE, 3 * FEAT), lambda i: (i, 0)),
            fspec, fspec, fspec,
            pl.BlockSpec((1, FEAT), lambda i: (0, 0)),
            pl.BlockSpec((FEAT, 3 * FEAT), lambda i: (0, 0)),
        ],
        out_specs=[fspec, fspec, fspec, fspec],
        out_shape=out4,
    )(r_ij, phi_g, vxg, vyg, vzg, freq, wrbf_pad)


_NW = 32
_EPW = N_EDGES // _NW
_CH = 80
_NCHUNK = _EPW // _CH


def _gather_body(phi_hbm, vx_hbm, vy_hbm, vz_hbm, dst_hbm,
                 phig_hbm, vxg_hbm, vyg_hbm, vzg_hbm,
                 idx_v, phi_b, vx_b, vy_b, vz_b, sem):
    wid = lax.axis_index("s") * 2 + lax.axis_index("c")
    base = wid * _EPW

    def chunk(j, _):
        off = base + j * _CH
        pltpu.sync_copy(dst_hbm.at[pl.ds(off, _CH)], idx_v)
        pltpu.async_copy(phi_hbm.at[idx_v], phi_b, sem).wait()
        pltpu.async_copy(vx_hbm.at[idx_v], vx_b, sem).wait()
        pltpu.async_copy(vy_hbm.at[idx_v], vy_b, sem).wait()
        pltpu.async_copy(vz_hbm.at[idx_v], vz_b, sem).wait()
        pltpu.sync_copy(phi_b, phig_hbm.at[pl.ds(off, _CH)])
        pltpu.sync_copy(vx_b, vxg_hbm.at[pl.ds(off, _CH)])
        pltpu.sync_copy(vy_b, vyg_hbm.at[pl.ds(off, _CH)])
        pltpu.sync_copy(vz_b, vzg_hbm.at[pl.ds(off, _CH)])
        return _

    lax.fori_loop(0, _NCHUNK, chunk, 0)


def _sc_gather(phi, vx, vy, vz, dst):
    mesh = plsc.VectorSubcoreMesh(core_axis_name="c", subcore_axis_name="s")
    e = dst.shape[0]
    out_type = [
        jax.ShapeDtypeStruct((e, 3 * FEAT), jnp.float32),
        jax.ShapeDtypeStruct((e, FEAT), jnp.float32),
        jax.ShapeDtypeStruct((e, FEAT), jnp.float32),
        jax.ShapeDtypeStruct((e, FEAT), jnp.float32),
    ]
    f = pl.kernel(
        _gather_body,
        out_type=out_type,
        mesh=mesh,
        scratch_types=[
            pltpu.VMEM((_CH,), jnp.int32),
            pltpu.VMEM((_CH, 3 * FEAT), jnp.float32),
            pltpu.VMEM((_CH, FEAT), jnp.float32),
            pltpu.VMEM((_CH, FEAT), jnp.float32),
            pltpu.VMEM((_CH, FEAT), jnp.float32),
            pltpu.SemaphoreType.DMA,
        ],
    )
    return f(phi, vx, vy, vz, dst)


_NT = 16
_EPT = N_EDGES // _NT
_NCH_S = _EPT // _CH
_FB = 80
_NFB = N_NODES // _FB


_NG = 5
_CPG = _NCH_S // _NG


def _scatter_body(ds_hbm, dvx_hbm, dvy_hbm, dvz_hbm, src4_hbm,
                  os_hbm, ovx_hbm, ovy_hbm, ovz_hbm,
                  acc, idx_buf, dbuf, zbuf):
    cid = lax.axis_index("c")
    sid = lax.axis_index("s")

    def zloop(k, carry):
        zbuf[k // 8, pl.ds((k % 8) * 16, 16)] = jnp.zeros((16,), jnp.float32)
        return carry

    lax.fori_loop(0, _FB * (FEAT // 16), zloop, 0)

    def one_pass(d_hbm, o_hbm):
        for t in range(-(-_NFB // _NT)):
            b = t * _NT + sid

            @pl.when(b < _NFB)
            def _():
                pltpu.sync_copy(zbuf, acc.at[pl.ds(b * _FB, _FB)])

        plsc.subcore_barrier()

        for g in range(_NG):
            pltpu.sync_copy(src4_hbm.at[sid, g], idx_buf)

            def chunk(j, carry, g=g):
                off = sid * _EPT + (g * _CPG + j) * _CH
                pltpu.sync_copy(d_hbm.at[pl.ds(off, _CH)], dbuf)
                pltpu.sync_copy(dbuf, acc.at[idx_buf.at[j]], add=True)
                return carry

            lax.fori_loop(0, _CPG, chunk, 0)
        plsc.subcore_barrier()
        for t in range(-(-_NFB // _NT)):
            b = t * _NT + sid

            @pl.when(b < _NFB)
            def _():
                rows = pl.ds(b * _FB, _FB)
                pltpu.sync_copy(acc.at[rows], o_hbm.at[rows])

        plsc.subcore_barrier()

    @pl.when(cid == 0)
    def _():
        one_pass(ds_hbm, os_hbm)
        one_pass(dvx_hbm, ovx_hbm)

    @pl.when(cid == 1)
    def _():
        one_pass(dvy_hbm, ovy_hbm)
        one_pass(dvz_hbm, ovz_hbm)


def _sc_scatter(ds, dvx, dvy, dvz, src):
    mesh = plsc.VectorSubcoreMesh(core_axis_name="c", subcore_axis_name="s")
    src4 = src.reshape(_NT, _NG, _CPG, _CH)
    out_type = [jax.ShapeDtypeStruct((N_NODES, FEAT), jnp.float32)] * 4
    f = pl.kernel(
        _scatter_body,
        out_type=out_type,
        mesh=mesh,
        scratch_types=[
            pltpu.VMEM_SHARED((N_NODES, FEAT), jnp.float32),
            pltpu.VMEM((_CPG, _CH), jnp.int32),
            pltpu.VMEM((_CH, FEAT), jnp.float32),
            pltpu.VMEM((_FB, FEAT), jnp.float32),
        ],
    )
    return f(ds, dvx, dvy, dvz, src4)


def kernel(s_j, v_j, r_ij, nbrs, W_phi, b_phi, W_rbf):
    nbrs = nbrs.astype(jnp.int32)
    src = nbrs[:, 0]
    dst = nbrs[:, 1]
    phi = _compute_phi(s_j, W_phi, b_phi)
    vt = jnp.transpose(v_j, (2, 0, 1))
    phi_g, vxg, vyg, vzg = _sc_gather(phi, vt[0], vt[1], vt[2], dst)
    ds, dvx, dvy, dvz = _edge_math(r_ij, phi_g, vxg, vyg, vzg, W_rbf)
    delta_s, ovx, ovy, ovz = _sc_scatter(ds, dvx, dvy, dvz, src)
    delta_v = jnp.stack([ovx, ovy, ovz], axis=-1)
    return (delta_s, delta_v)


# speedup vs baseline: 16.1366x; 1.2906x over previous
import functools

import jax
import jax.numpy as jnp
from jax import lax
from jax.experimental import pallas as pl
from jax.experimental.pallas import tpu as pltpu
from jax.experimental.pallas import tpu_sc as plsc

EPS = 1e-15
N_NODES = 10000
N_EDGES = 320000
FEAT = 128
N_RBF = 20
CUTOFF = 5.0


_BN = 1000


def _phi_body(s_ref, w_ref, b_ref, o_ref):
    o_ref[...] = (
        jnp.dot(s_ref[...], w_ref[...], preferred_element_type=jnp.float32)
        + b_ref[...]
    )


def _compute_phi(s_j, W_phi, b_phi):
    n = s_j.shape[0]
    grid = n // _BN
    return pl.pallas_call(
        _phi_body,
        grid=(grid,),
        in_specs=[
            pl.BlockSpec((_BN, FEAT), lambda i: (i, 0)),
            pl.BlockSpec((FEAT, 3 * FEAT), lambda i: (0, 0)),
            pl.BlockSpec((1, 3 * FEAT), lambda i: (0, 0)),
        ],
        out_specs=pl.BlockSpec((_BN, 3 * FEAT), lambda i: (i, 0)),
        out_shape=jax.ShapeDtypeStruct((n, 3 * FEAT), jnp.float32),
    )(s_j, W_phi, b_phi.reshape(1, -1))


_BE = 1000


def _edge_body(r_ref, tabg_ref, freq_ref, wrbf_ref,
               ds_ref, dvx_ref, dvy_ref, dvz_ref):
    r = r_ref[...]
    d2 = (r * r).sum(axis=1, keepdims=True) + 3.0 * EPS
    dist = jnp.sqrt(d2)
    inv = 1.0 / dist
    rbf = jnp.sin(dist * freq_ref[...]) * inv
    w_s = jnp.dot(rbf, wrbf_ref[...], preferred_element_type=jnp.float32,
                  precision=jax.lax.Precision.HIGHEST)
    t = tabg_ref[...]
    sp0 = t[:, :FEAT] * w_s[:, :FEAT]
    sp1 = t[:, FEAT:2 * FEAT] * w_s[:, FEAT:2 * FEAT]
    sp2 = t[:, 2 * FEAT:3 * FEAT] * w_s[:, 2 * FEAT:]
    ds_ref[...] = sp1
    ux = r[:, 0:1] * inv
    uy = r[:, 1:2] * inv
    uz = r[:, 2:3] * inv
    dvx_ref[...] = sp2 * ux + sp0 * t[:, 3 * FEAT:4 * FEAT]
    dvy_ref[...] = sp2 * uy + sp0 * t[:, 4 * FEAT:5 * FEAT]
    dvz_ref[...] = sp2 * uz + sp0 * t[:, 5 * FEAT:]


def _edge_math(r_ij, tabg, W_rbf):
    e = r_ij.shape[0]
    grid = e // _BE
    freq = jnp.pad(
        jnp.arange(1, N_RBF + 1, dtype=jnp.float32) * (jnp.pi / CUTOFF),
        (0, FEAT - N_RBF),
    ).reshape(1, FEAT)
    wrbf_pad = jnp.pad(W_rbf, ((0, FEAT - N_RBF), (0, 0)))
    fspec = pl.BlockSpec((_BE, FEAT), lambda i: (i, 0))
    out4 = [jax.ShapeDtypeStruct((e, FEAT), jnp.float32)] * 4
    return pl.pallas_call(
        _edge_body,
        grid=(grid,),
        in_specs=[
            pl.BlockSpec((_BE, 3), lambda i: (i, 0)),
            pl.BlockSpec((_BE, _TABW), lambda i: (i, 0)),
            pl.BlockSpec((1, FEAT), lambda i: (0, 0)),
            pl.BlockSpec((FEAT, 3 * FEAT), lambda i: (0, 0)),
        ],
        out_specs=[fspec, fspec, fspec, fspec],
        out_shape=out4,
    )(r_ij, tabg, freq, wrbf_pad)


_NW = 32
_EPW = N_EDGES // _NW
_CH = 80
_GCH = 40
_GNCH = _EPW // _GCH
_TABW = 6 * FEAT


def _gather_body(tab_hbm, dst3_hbm, tabg_hbm,
                 idx_all, buf0, buf1, sem0, sem1):
    wid = lax.axis_index("s") * 2 + lax.axis_index("c")
    base = wid * _EPW
    pltpu.sync_copy(dst3_hbm.at[wid], idx_all)

    dummy = tab_hbm.at[pl.ds(0, _GCH)]
    pltpu.async_copy(tab_hbm.at[idx_all.at[0]], buf0, sem0)

    def pair(p, carry):
        j0 = 2 * p
        j1 = j0 + 1
        pltpu.async_copy(tab_hbm.at[idx_all.at[j1]], buf1, sem1)
        pltpu.make_async_copy(dummy, buf0, sem0).wait()
        pltpu.sync_copy(buf0, tabg_hbm.at[pl.ds(base + j0 * _GCH, _GCH)])

        @pl.when(j1 + 1 < _GNCH)
        def _():
            pltpu.async_copy(tab_hbm.at[idx_all.at[j1 + 1]], buf0, sem0)

        pltpu.make_async_copy(dummy, buf1, sem1).wait()
        pltpu.sync_copy(buf1, tabg_hbm.at[pl.ds(base + j1 * _GCH, _GCH)])
        return carry

    lax.fori_loop(0, _GNCH // 2, pair, 0)


def _sc_gather(tab, dst):
    mesh = plsc.VectorSubcoreMesh(core_axis_name="c", subcore_axis_name="s")
    e = dst.shape[0]
    dst3 = dst.reshape(_NW, _GNCH, _GCH)
    out_type = jax.ShapeDtypeStruct((e, _TABW), jnp.float32)
    f = pl.kernel(
        _gather_body,
        out_type=out_type,
        mesh=mesh,
        scratch_types=[
            pltpu.VMEM((_GNCH, _GCH), jnp.int32),
            pltpu.VMEM((_GCH, _TABW), jnp.float32),
            pltpu.VMEM((_GCH, _TABW), jnp.float32),
            pltpu.SemaphoreType.DMA,
            pltpu.SemaphoreType.DMA,
        ],
    )
    return f(tab, dst3)


_NT = 16
_EPT = N_EDGES // _NT
_NCH_S = _EPT // _CH
_FB = 80
_NFB = N_NODES // _FB


_NG = 5
_CPG = _NCH_S // _NG


def _scatter_body(ds_hbm, dvx_hbm, dvy_hbm, dvz_hbm, src4_hbm,
                  os_hbm, ovx_hbm, ovy_hbm, ovz_hbm,
                  acc, idx_buf, dbuf0, dbuf1, sem0, sem1):
    cid = lax.axis_index("c")
    sid = lax.axis_index("s")

    def one_pass(d_hbm, o_hbm):
        def zloop(k, carry):
            dbuf0[k // 8, pl.ds((k % 8) * 16, 16)] = jnp.zeros((16,),
                                                               jnp.float32)
            return carry

        lax.fori_loop(0, _FB * (FEAT // 16), zloop, 0)
        for t in range(-(-_NFB // _NT)):
            b = t * _NT + sid

            @pl.when(b < _NFB)
            def _():
                pltpu.sync_copy(dbuf0, acc.at[pl.ds(b * _FB, _FB)])

        plsc.subcore_barrier()

        dummy = d_hbm.at[pl.ds(0, _CH)]
        for g in range(_NG):
            pltpu.sync_copy(src4_hbm.at[sid, g], idx_buf)
            gbase = sid * _EPT + g * _CPG * _CH
            pltpu.async_copy(d_hbm.at[pl.ds(gbase, _CH)], dbuf0, sem0)

            def pair(p, carry, gbase=gbase):
                j0 = 2 * p
                j1 = j0 + 1
                pltpu.async_copy(d_hbm.at[pl.ds(gbase + j1 * _CH, _CH)],
                                 dbuf1, sem1)
                pltpu.make_async_copy(dummy, dbuf0, sem0).wait()
                pltpu.sync_copy(dbuf0, acc.at[idx_buf.at[j0]], add=True)

                @pl.when(j1 + 1 < _CPG)
                def _():
                    pltpu.async_copy(
                        d_hbm.at[pl.ds(gbase + (j1 + 1) * _CH, _CH)],
                        dbuf0, sem0)

                pltpu.make_async_copy(dummy, dbuf1, sem1).wait()
                pltpu.sync_copy(dbuf1, acc.at[idx_buf.at[j1]], add=True)
                return carry

            lax.fori_loop(0, _CPG // 2, pair, 0)
        plsc.subcore_barrier()
        for t in range(-(-_NFB // _NT)):
            b = t * _NT + sid

            @pl.when(b < _NFB)
            def _():
                rows = pl.ds(b * _FB, _FB)
                pltpu.sync_copy(acc.at[rows], o_hbm.at[rows])

        plsc.subcore_barrier()

    @pl.when(cid == 0)
    def _():
        one_pass(ds_hbm, os_hbm)
        one_pass(dvx_hbm, ovx_hbm)

    @pl.when(cid == 1)
    def _():
        one_pass(dvy_hbm, ovy_hbm)
        one_pass(dvz_hbm, ovz_hbm)


def _sc_scatter(ds, dvx, dvy, dvz, src):
    mesh = plsc.VectorSubcoreMesh(core_axis_name="c", subcore_axis_name="s")
    src4 = src.reshape(_NT, _NG, _CPG, _CH)
    out_type = [jax.ShapeDtypeStruct((N_NODES, FEAT), jnp.float32)] * 4
    f = pl.kernel(
        _scatter_body,
        out_type=out_type,
        mesh=mesh,
        scratch_types=[
            pltpu.VMEM_SHARED((N_NODES, FEAT), jnp.float32),
            pltpu.VMEM((_CPG, _CH), jnp.int32),
            pltpu.VMEM((_CH, FEAT), jnp.float32),
            pltpu.VMEM((_CH, FEAT), jnp.float32),
            pltpu.SemaphoreType.DMA,
            pltpu.SemaphoreType.DMA,
        ],
    )
    return f(ds, dvx, dvy, dvz, src4)


def kernel(s_j, v_j, r_ij, nbrs, W_phi, b_phi, W_rbf):
    nbrs = nbrs.astype(jnp.int32)
    src = nbrs[:, 0]
    dst = nbrs[:, 1]
    phi = _compute_phi(s_j, W_phi, b_phi)
    vt = jnp.transpose(v_j, (2, 0, 1))
    tab = jnp.concatenate([phi, vt[0], vt[1], vt[2]], axis=1)
    tabg = _sc_gather(tab, dst)
    ds, dvx, dvy, dvz = _edge_math(r_ij, tabg, W_rbf)
    delta_s, ovx, ovy, ovz = _sc_scatter(ds, dvx, dvy, dvz, src)
    delta_v = jnp.stack([ovx, ovy, ovz], axis=-1)
    return (delta_s, delta_v)
